# baseline (speedup 1.0000x reference)
import functools

import jax
import jax.numpy as jnp
from jax import lax
from jax.experimental import pallas as pl
from jax.experimental.pallas import tpu as pltpu
from jax.experimental.pallas import tpu_sc as plsc

N = 10000
E = 320000
F = 128
H = 64

NC = 2
NS = 16
NW = NC * NS

NSPLIT = 2
EH = E // NSPLIT
EPW = EH // NW
GB = 40
NG = EPW // GB
KG = 5
NJ = NG // KG
NPW = 1000

_mesh = functools.partial(
    plsc.VectorSubcoreMesh, core_axis_name="c", subcore_axis_name="s")


def _gather_body(h_hbm, srcr_hbm, dstr_hbm, ga_hbm, gb_hbm,
                 idx, buf_a, buf_b, sem_a, sem_b):
    cid = lax.axis_index("c")
    sid = lax.axis_index("s")
    wid = sid * NC + cid
    e_base = wid * EPW

    def phase(idxr_hbm, out_hbm):
        pltpu.sync_copy(idxr_hbm.at[wid], idx)

        def fire(j, buf, sem):
            for k in range(KG):
                pltpu.async_copy(h_hbm.at[idx.at[j * KG + k]],
                                 buf.at[pl.ds(k * GB, GB)], sem)

        def drain(j, buf, sem):
            for k in range(KG):
                pltpu.make_async_copy(h_hbm.at[idx.at[j * KG + k]],
                                      buf.at[pl.ds(k * GB, GB)], sem).wait()

        def store(j, buf):
            pltpu.sync_copy(buf, out_hbm.at[pl.ds(e_base + j * (KG * GB), KG * GB)])

        fire(0, buf_a, sem_a)

        def body(t, _):
            ja = 2 * t
            drain(ja, buf_a, sem_a)
            fire(ja + 1, buf_b, sem_b)
            store(ja, buf_a)
            drain(ja + 1, buf_b, sem_b)
            fire(ja + 2, buf_a, sem_a)
            store(ja + 1, buf_b)
            return 0

        lax.fori_loop(0, (NJ - 1) // 2, body, 0)
        drain(NJ - 1, buf_a, sem_a)
        store(NJ - 1, buf_a)

    phase(srcr_hbm, ga_hbm)
    phase(dstr_hbm, gb_hbm)


def _sc_gather(h, srcr, dstr):
    return pl.kernel(
        _gather_body,
        out_type=[
            jax.ShapeDtypeStruct((EH, F), jnp.float32),
            jax.ShapeDtypeStruct((EH, F), jnp.float32),
        ],
        mesh=_mesh(),
        scratch_types=[
            pltpu.VMEM((NG, GB), jnp.int32),
            pltpu.VMEM((KG * GB, F), jnp.float32),
            pltpu.VMEM((KG * GB, F), jnp.float32),
            pltpu.SemaphoreType.DMA,
            pltpu.SemaphoreType.DMA,
        ],
    )(h, srcr, dstr)


def _scatter_body(m_hbm, dstr_hbm, zeros_hbm, upd_hbm,
                  shared, idx_d, buf_a, buf_b, sem_a, sem_b):
    cid = lax.axis_index("c")
    sid = lax.axis_index("s")
    wid = sid * NC + cid
    e_base = wid * EPW
    @pl.when(sid < N // NPW)
    def _():
        pltpu.sync_copy(zeros_hbm.at[pl.ds(sid * NPW, NPW)],
                        shared.at[pl.ds(sid * NPW, NPW)])
    pltpu.sync_copy(dstr_hbm.at[wid], idx_d)
    plsc.subcore_barrier()

    def fire(j, buf, sem):
        pltpu.async_copy(m_hbm.at[pl.ds(e_base + j * GB, GB)], buf, sem)

    def drain(j, buf, sem):
        pltpu.make_async_copy(m_hbm.at[pl.ds(e_base + j * GB, GB)], buf, sem).wait()

    def scat(j, buf):
        pltpu.sync_copy(buf, shared.at[idx_d.at[j]], add=True)

    fire(0, buf_a, sem_a)

    def body(t, _):
        ja = 2 * t
        drain(ja, buf_a, sem_a)
        fire(ja + 1, buf_b, sem_b)
        scat(ja, buf_a)
        drain(ja + 1, buf_b, sem_b)
        fire(ja + 2, buf_a, sem_a)
        scat(ja + 1, buf_b)
        return 0

    lax.fori_loop(0, (NG - 1) // 2, body, 0)
    drain(NG - 1, buf_a, sem_a)
    scat(NG - 1, buf_a)
    plsc.subcore_barrier()

    @pl.when(sid < N // NPW)
    def _():
        pltpu.sync_copy(shared.at[pl.ds(sid * NPW, NPW)],
                        upd_hbm.at[pl.ds(cid * N + sid * NPW, NPW)])


def _sc_scatter(m, dstr, zeros):
    return pl.kernel(
        _scatter_body,
        out_type=jax.ShapeDtypeStruct((2 * N, F), jnp.float32),
        mesh=_mesh(),
        scratch_types=[
            pltpu.VMEM_SHARED((N, F), jnp.float32),
            pltpu.VMEM((NG, GB), jnp.int32),
            pltpu.VMEM((GB, F), jnp.float32),
            pltpu.VMEM((GB, F), jnp.float32),
            pltpu.SemaphoreType.DMA,
            pltpu.SemaphoreType.DMA,
        ],
    )(m, dstr, zeros)


def _edge_body(ga_ref, gb_ref, ef_ref, ws_ref, bs_ref, wd_ref, bd_ref,
               w1_ref, b1_ref, w2_ref, b2_ref, w3_ref, b3_ref, m_ref):
    a = ga_ref[...]
    sc = jnp.dot(a, ws_ref[...], preferred_element_type=jnp.float32) + bs_ref[...]
    dc = jnp.dot(gb_ref[...], wd_ref[...], preferred_element_type=jnp.float32) + bd_ref[...]
    t = jax.nn.relu(ef_ref[...] + sc + dc)
    t = jax.nn.relu(jnp.dot(t, w1_ref[...], preferred_element_type=jnp.float32) + b1_ref[...])
    t = jax.nn.relu(jnp.dot(t, w2_ref[...], preferred_element_type=jnp.float32) + b2_ref[...])
    e = jnp.dot(t, w3_ref[...], preferred_element_type=jnp.float32) + b3_ref[...]
    m_ref[...] = a * e


def _node_body(h_ref, u0_ref, u1_ref, u2_ref, u3_ref,
               wpd_ref, bpd_ref, wpu_ref, bpu_ref,
               wt1_ref, bt1_ref, wt2_ref, bt2_ref, out_ref):
    u = (u0_ref[...] + u1_ref[...]) + (u2_ref[...] + u3_ref[...])
    pre = (jnp.dot(h_ref[...], wpd_ref[...], preferred_element_type=jnp.float32)
           + bpd_ref[...]
           + jnp.dot(u, wpu_ref[...], preferred_element_type=jnp.float32)
           + bpu_ref[...])
    z = jax.nn.relu(pre)
    z = jax.nn.relu(jnp.dot(z, wt1_ref[...], preferred_element_type=jnp.float32)
                    + bt1_ref[...])
    out_ref[...] = (jnp.dot(z, wt2_ref[...], preferred_element_type=jnp.float32)
                    + bt2_ref[...])


def _full(shape):
    return pl.BlockSpec(shape, lambda i: (0, 0))


def _edge_mlp(ga, gb, ef, W_src, b_src, W_dst, b_dst,
              Wp1, bp1, Wp2, bp2, Wp3, bp3):
    eb = 2000
    return pl.pallas_call(
        _edge_body,
        grid=(EH // eb,),
        in_specs=[
            pl.BlockSpec((eb, F), lambda i: (i, 0)),
            pl.BlockSpec((eb, F), lambda i: (i, 0)),
            pl.BlockSpec((eb, H), lambda i: (i, 0)),
            _full((F, H)), _full((1, H)),
            _full((F, H)), _full((1, H)),
            _full((H, H)), _full((1, H)),
            _full((H, H)), _full((1, H)),
            _full((H, F)), _full((1, F)),
        ],
        out_specs=pl.BlockSpec((eb, F), lambda i: (i, 0)),
        out_shape=jax.ShapeDtypeStruct((EH, F), jnp.float32),
    )(ga, gb, ef, W_src, b_src.reshape(1, H), W_dst, b_dst.reshape(1, H),
      Wp1, bp1.reshape(1, H), Wp2, bp2.reshape(1, H), Wp3, bp3.reshape(1, F))


def kernel(node_feat, edge_index, edge_feat, W_src, b_src, W_dst, b_dst,
           Wp1, bp1, Wp2, bp2, Wp3, bp3, Wpd, bpd, Wpu, bpu,
           Wt1, bt1, Wt2, bt2):
    f32 = jnp.float32
    src = edge_index[0].reshape(NSPLIT, NW, NG, GB)
    dst = edge_index[1].reshape(NSPLIT, NW, NG, GB)
    efs = edge_feat.reshape(NSPLIT, EH, H)
    zeros = jnp.zeros((N, F), f32)

    upds = []
    edge_args = (W_src, b_src, W_dst, b_dst, Wp1, bp1, Wp2, bp2, Wp3, bp3)
    for s in range(NSPLIT):
        ga, gb = _sc_gather(node_feat, src[s], dst[s])
        m = _edge_mlp(ga, gb, efs[s], *edge_args)
        upds.append(_sc_scatter(m, dst[s], zeros))

    nb = 1000
    nblocks = N // nb
    u_specs = [pl.BlockSpec((nb, F), lambda i, o=off: (i + o, 0))
               for off in (0, nblocks, 0, nblocks)]
    out = pl.pallas_call(
        _node_body,
        grid=(nblocks,),
        in_specs=[pl.BlockSpec((nb, F), lambda i: (i, 0))] + u_specs + [
            _full((F, H)), _full((1, H)),
            _full((F, H)), _full((1, H)),
            _full((H, F)), _full((1, F)),
            _full((F, F)), _full((1, F)),
        ],
        out_specs=pl.BlockSpec((nb, F), lambda i: (i, 0)),
        out_shape=jax.ShapeDtypeStruct((N, F), f32),
    )(node_feat, upds[0], upds[0], upds[1], upds[1],
      Wpd, bpd.reshape(1, H), Wpu, bpu.reshape(1, H),
      Wt1, bt1.reshape(1, F), Wt2, bt2.reshape(1, F))
    return out


# speedup vs baseline: 1.0993x; 1.0993x over previous
import functools

import jax
import jax.numpy as jnp
from jax import lax
from jax.experimental import pallas as pl
from jax.experimental.pallas import tpu as pltpu
from jax.experimental.pallas import tpu_sc as plsc

N = 10000
E = 320000
F = 128
H = 64

NC = 2
NS = 16
NW = NC * NS

NSPLIT = 2
EH = E // NSPLIT
EPW = EH // NW
GB = 100
NG = EPW // GB
KG = 2
NJ = NG // KG
GBS = 40
NGS = EPW // GBS
NPW = 1000

_mesh = functools.partial(
    plsc.VectorSubcoreMesh, core_axis_name="c", subcore_axis_name="s")


def _gather_body(s, h_hbm, srcr_hbm, dstr_hbm, ga_hbm, gb_hbm,
                 idx, buf_a, buf_b, sem_a, sem_b):
    cid = lax.axis_index("c")
    sid = lax.axis_index("s")
    wid = sid * NC + cid
    e_base = wid * EPW

    def phase(idxr_hbm, out_hbm):
        pltpu.sync_copy(idxr_hbm.at[s * NW + wid], idx)

        def fire(j, buf, sem):
            for k in range(KG):
                pltpu.async_copy(h_hbm.at[idx.at[j * KG + k]],
                                 buf.at[pl.ds(k * GB, GB)], sem)

        def drain(j, buf, sem):
            for k in range(KG):
                pltpu.make_async_copy(h_hbm.at[idx.at[j * KG + k]],
                                      buf.at[pl.ds(k * GB, GB)], sem).wait()

        def store(j, buf):
            pltpu.sync_copy(buf, out_hbm.at[pl.ds(e_base + j * (KG * GB), KG * GB)])

        fire(0, buf_a, sem_a)

        def body(t, _):
            ja = 2 * t
            drain(ja, buf_a, sem_a)
            fire(ja + 1, buf_b, sem_b)
            store(ja, buf_a)
            drain(ja + 1, buf_b, sem_b)
            fire(ja + 2, buf_a, sem_a)
            store(ja + 1, buf_b)
            return 0

        lax.fori_loop(0, (NJ - 1) // 2, body, 0)
        drain(NJ - 1, buf_a, sem_a)
        store(NJ - 1, buf_a)

    phase(srcr_hbm, ga_hbm)
    phase(dstr_hbm, gb_hbm)


def _sc_gather(s, h, srcr, dstr):
    return pl.kernel(
        functools.partial(_gather_body, s),
        out_type=[
            jax.ShapeDtypeStruct((EH, F), jnp.float32),
            jax.ShapeDtypeStruct((EH, F), jnp.float32),
        ],
        mesh=_mesh(),
        scratch_types=[
            pltpu.VMEM((NG, GB), jnp.int32),
            pltpu.VMEM((KG * GB, F), jnp.float32),
            pltpu.VMEM((KG * GB, F), jnp.float32),
            pltpu.SemaphoreType.DMA,
            pltpu.SemaphoreType.DMA,
        ],
    )(h, srcr, dstr)


def _scatter_body(s, m_hbm, dstr_hbm, zeros_hbm, upd_hbm,
                  shared, idx_d, buf_a, buf_b, sem_a, sem_b):
    cid = lax.axis_index("c")
    sid = lax.axis_index("s")
    wid = sid * NC + cid
    e_base = wid * EPW
    @pl.when(sid < N // NPW)
    def _():
        pltpu.sync_copy(zeros_hbm.at[pl.ds(sid * NPW, NPW)],
                        shared.at[pl.ds(sid * NPW, NPW)])
    pltpu.sync_copy(dstr_hbm.at[s * NW + wid], idx_d)
    plsc.subcore_barrier()

    def fire(j, buf, sem):
        pltpu.async_copy(m_hbm.at[pl.ds(e_base + j * GBS, GBS)], buf, sem)

    def drain(j, buf, sem):
        pltpu.make_async_copy(m_hbm.at[pl.ds(e_base + j * GBS, GBS)], buf, sem).wait()

    def scat(j, buf):
        pltpu.sync_copy(buf, shared.at[idx_d.at[j]], add=True)

    fire(0, buf_a, sem_a)

    def body(t, _):
        ja = 2 * t
        drain(ja, buf_a, sem_a)
        fire(ja + 1, buf_b, sem_b)
        scat(ja, buf_a)
        drain(ja + 1, buf_b, sem_b)
        fire(ja + 2, buf_a, sem_a)
        scat(ja + 1, buf_b)
        return 0

    lax.fori_loop(0, (NGS - 1) // 2, body, 0)
    drain(NGS - 1, buf_a, sem_a)
    scat(NGS - 1, buf_a)
    plsc.subcore_barrier()

    @pl.when(sid < N // NPW)
    def _():
        pltpu.sync_copy(shared.at[pl.ds(sid * NPW, NPW)],
                        upd_hbm.at[pl.ds(cid * N + sid * NPW, NPW)])


def _sc_scatter(s, m, dstr, zeros):
    return pl.kernel(
        functools.partial(_scatter_body, s),
        out_type=jax.ShapeDtypeStruct((2 * N, F), jnp.float32),
        mesh=_mesh(),
        scratch_types=[
            pltpu.VMEM_SHARED((N, F), jnp.float32),
            pltpu.VMEM((NGS, GBS), jnp.int32),
            pltpu.VMEM((GBS, F), jnp.float32),
            pltpu.VMEM((GBS, F), jnp.float32),
            pltpu.SemaphoreType.DMA,
            pltpu.SemaphoreType.DMA,
        ],
    )(m, dstr, zeros)


def _edge_body(ga_ref, gb_ref, ef_ref, ws_ref, bs_ref, wd_ref, bd_ref,
               w1_ref, b1_ref, w2_ref, b2_ref, w3_ref, b3_ref, m_ref):
    a = ga_ref[...]
    sc = jnp.dot(a, ws_ref[...], preferred_element_type=jnp.float32) + bs_ref[...]
    dc = jnp.dot(gb_ref[...], wd_ref[...], preferred_element_type=jnp.float32) + bd_ref[...]
    t = jax.nn.relu(ef_ref[...] + sc + dc)
    t = jax.nn.relu(jnp.dot(t, w1_ref[...], preferred_element_type=jnp.float32) + b1_ref[...])
    t = jax.nn.relu(jnp.dot(t, w2_ref[...], preferred_element_type=jnp.float32) + b2_ref[...])
    e = jnp.dot(t, w3_ref[...], preferred_element_type=jnp.float32) + b3_ref[...]
    m_ref[...] = a * e


def _node_body(h_ref, u0_ref, u1_ref, u2_ref, u3_ref,
               wpd_ref, bpd_ref, wpu_ref, bpu_ref,
               wt1_ref, bt1_ref, wt2_ref, bt2_ref, out_ref):
    u = (u0_ref[...] + u1_ref[...]) + (u2_ref[...] + u3_ref[...])
    pre = (jnp.dot(h_ref[...], wpd_ref[...], preferred_element_type=jnp.float32)
           + bpd_ref[...]
           + jnp.dot(u, wpu_ref[...], preferred_element_type=jnp.float32)
           + bpu_ref[...])
    z = jax.nn.relu(pre)
    z = jax.nn.relu(jnp.dot(z, wt1_ref[...], preferred_element_type=jnp.float32)
                    + bt1_ref[...])
    out_ref[...] = (jnp.dot(z, wt2_ref[...], preferred_element_type=jnp.float32)
                    + bt2_ref[...])


def _full(shape):
    return pl.BlockSpec(shape, lambda i: (0, 0))


def _edge_mlp(s, ga, gb, ef, W_src, b_src, W_dst, b_dst,
              Wp1, bp1, Wp2, bp2, Wp3, bp3):
    eb = 2000
    off = s * (EH // eb)
    return pl.pallas_call(
        _edge_body,
        grid=(EH // eb,),
        in_specs=[
            pl.BlockSpec((eb, F), lambda i: (i, 0)),
            pl.BlockSpec((eb, F), lambda i: (i, 0)),
            pl.BlockSpec((eb, H), lambda i: (i + off, 0)),
            _full((F, H)), _full((1, H)),
            _full((F, H)), _full((1, H)),
            _full((H, H)), _full((1, H)),
            _full((H, H)), _full((1, H)),
            _full((H, F)), _full((1, F)),
        ],
        out_specs=pl.BlockSpec((eb, F), lambda i: (i, 0)),
        out_shape=jax.ShapeDtypeStruct((EH, F), jnp.float32),
    )(ga, gb, ef, W_src, b_src.reshape(1, H), W_dst, b_dst.reshape(1, H),
      Wp1, bp1.reshape(1, H), Wp2, bp2.reshape(1, H), Wp3, bp3.reshape(1, F))


def kernel(node_feat, edge_index, edge_feat, W_src, b_src, W_dst, b_dst,
           Wp1, bp1, Wp2, bp2, Wp3, bp3, Wpd, bpd, Wpu, bpu,
           Wt1, bt1, Wt2, bt2):
    f32 = jnp.float32
    srcg = edge_index[0].reshape(NSPLIT * NW, NG, GB)
    dstg = edge_index[1].reshape(NSPLIT * NW, NG, GB)
    dsts = edge_index[1].reshape(NSPLIT * NW, NGS, GBS)
    zeros = jnp.zeros((N, F), f32)

    upds = []
    edge_args = (W_src, b_src, W_dst, b_dst, Wp1, bp1, Wp2, bp2, Wp3, bp3)
    for s in range(NSPLIT):
        ga, gb = _sc_gather(s, node_feat, srcg, dstg)
        m = _edge_mlp(s, ga, gb, edge_feat, *edge_args)
        upds.append(_sc_scatter(s, m, dsts, zeros))

    nb = 1000
    nblocks = N // nb
    u_specs = [pl.BlockSpec((nb, F), lambda i, o=off: (i + o, 0))
               for off in (0, nblocks, 0, nblocks)]
    out = pl.pallas_call(
        _node_body,
        grid=(nblocks,),
        in_specs=[pl.BlockSpec((nb, F), lambda i: (i, 0))] + u_specs + [
            _full((F, H)), _full((1, H)),
            _full((F, H)), _full((1, H)),
            _full((H, F)), _full((1, F)),
            _full((F, F)), _full((1, F)),
        ],
        out_specs=pl.BlockSpec((nb, F), lambda i: (i, 0)),
        out_shape=jax.ShapeDtypeStruct((N, F), f32),
    )(node_feat, upds[0], upds[0], upds[1], upds[1],
      Wpd, bpd.reshape(1, H), Wpu, bpu.reshape(1, H),
      Wt1, bt1.reshape(1, F), Wt2, bt2.reshape(1, F))
    return out
